# baseline (speedup 1.0000x reference)
import functools

import jax
import jax.numpy as jnp
from jax import lax
from jax.experimental import pallas as pl
from jax.experimental.pallas import tpu as pltpu
from jax.experimental.pallas import tpu_sc as plsc

NU, NI = 4000, 6000
N = NU + NI
NP = 10240
K = 128
NE = 320000
NSUB = 16
CH = 80
NCHUNK = (NE // (2 * NSUB)) // CH
BATCH = 4096
LW = 1e-4

_mesh = plsc.VectorSubcoreMesh(core_axis_name="c", subcore_axis_name="s")
_params = pltpu.CompilerParams(needs_layout_passes=False,
                               use_tc_tiling_on_sc=False)


def _vec(v, dtype=jnp.float32):
    return jnp.full((16,), v, dtype)


def _dinv16(d):
    x = jnp.maximum(d, _vec(1.0))
    i = lax.bitcast_convert_type(x, jnp.int32)
    i = _vec(0x5F3759DF, jnp.int32) - lax.shift_right_logical(i, _vec(1, jnp.int32))
    y = lax.bitcast_convert_type(i, jnp.float32)
    half = _vec(0.5) * x
    c15 = _vec(1.5)
    for _ in range(3):
        y = y * (c15 - half * y * y)
    return jnp.where(d > _vec(0.0), y, _vec(0.0))


def _mega_body(srcg_h, dstl_h, x0_h, zdeg_h, z375_h, iden_h, idxg_h, idxl_h,
               G_h, dva_h, y0_h, y1_h, y2_h,
               srcv, dstv, rowsA, rowsB, hist_v, iden_v, dinvv, deg40,
               gidx, dvbuf, acc_s, deg_s, dinv_s, semA, semB):
    c = lax.axis_index("c")
    s = lax.axis_index("s")
    pltpu.sync_copy(iden_h, iden_v)

    def _splat(w):
        return plsc.load_gather(
            dinvv, [jnp.full((16,), w // 16, jnp.int32),
                    jnp.full((16,), w % 16, jnp.int32)])

    def _z(i, carry):
        hist_v[i] = jnp.zeros((16,), jnp.float32)
        return carry
    lax.fori_loop(0, NP // 16, _z, 0)

    ones = _vec(1.0)
    for half in range(2):
        pltpu.sync_copy(dstl_h.at[half, s], dstv)
        off = _vec(NU if half == 0 else 0, jnp.int32)

        def _h(j, carry):
            def _hh(k, c2):
                idx = dstv[j, pl.ds(k * 16, 16)] + off
                hi = lax.shift_right_logical(idx, _vec(4, jnp.int32))
                lo = jnp.bitwise_and(idx, _vec(15, jnp.int32))
                plsc.addupdate_scatter(hist_v, [hi, lo], ones)
                return c2
            lax.fori_loop(0, CH // 16, _hh, 0)
            return carry
        lax.fori_loop(0, NCHUNK, _h, 0)

    pltpu.sync_copy(zdeg_h.at[pl.ds(s * 40, 40)], deg_s.at[pl.ds(s * 40, 40)])
    plsc.subcore_barrier()
    for r in range(5):
        pltpu.sync_copy(hist_v.at[pl.ds(r * 128, 128)],
                        deg_s.at[iden_v.at[r]], add=True)
    plsc.subcore_barrier()

    pltpu.sync_copy(deg_s.at[pl.ds(s * 40, 40)], deg40)

    def _dv(i, carry):
        deg40[i] = _dinv16(deg40[i])
        return carry
    lax.fori_loop(0, 40, _dv, 0)
    pltpu.sync_copy(deg40, dinv_s.at[pl.ds(s * 40, 40)])
    plsc.subcore_barrier()
    pltpu.sync_copy(dinv_s, dinvv)

    def _dvals(g):
        pltpu.sync_copy(idxg_h.at[g, pl.ds(s * 4, 4)], gidx)

        def _dvb(rk, carry):
            i16 = gidx[rk // 4, pl.ds((rk % 4) * 16, 16)]
            hi = lax.shift_right_logical(i16, _vec(4, jnp.int32))
            lo = jnp.bitwise_and(i16, _vec(15, jnp.int32))
            dvbuf[rk] = plsc.load_gather(dinvv, [hi, lo])
            return carry
        lax.fori_loop(0, 16, _dvb, 0)
        pltpu.sync_copy(dvbuf, dva_h.at[g, pl.ds(s * 16, 16)])

    @pl.when(c == 0)
    def _dvals_items():
        _dvals(1)
        _dvals(2)

    @pl.when(c == 1)
    def _dvals_users():
        _dvals(0)

    for g in range(3):
        pltpu.sync_copy(idxg_h.at[g, pl.ds(c * 32 + s * 2, 2)],
                        gidx.at[pl.ds(0, 2)])
        for kk in range(2):
            pltpu.async_copy(x0_h.at[gidx.at[kk]],
                             rowsA.at[pl.ds(0, 64)], semA).wait()
            pltpu.sync_copy(rowsA.at[pl.ds(0, 64)],
                            G_h.at[g, pl.ds(c * 2048 + s * 128 + kk * 64, 64)])

    def _scale_rows(src_h, dst_h, base, nchunks, chrows):
        for kk in range(nchunks):
            row0 = base + kk * chrows
            pltpu.sync_copy(src_h.at[pl.ds(row0, chrows)],
                            rowsA.at[pl.ds(0, chrows)])

            def _r(rr, c2):
                d16 = _splat(row0 + rr)

                def _k(kcol, c3):
                    sl = pl.ds(kcol * 16, 16)
                    rowsB[rr, sl] = rowsA[rr, sl] * d16
                    return c3
                lax.fori_loop(0, 8, _k, 0)
                return c2
            lax.fori_loop(0, chrows, _r, 0)
            pltpu.sync_copy(rowsB.at[pl.ds(0, chrows)],
                            dst_h.at[pl.ds(row0, chrows)])

    @pl.when(c == 0)
    def _y0_users():
        _scale_rows(x0_h, y0_h, s * 250, 5, 50)

    @pl.when(c == 1)
    def _y0_items():
        _scale_rows(x0_h, y0_h, NU + s * 375, 5, 75)

    plsc.subcore_barrier()

    ys = (y0_h, y1_h, y2_h)
    for t in range(3):
        ysrc = ys[t]

        half = (t + c) % 2
        pltpu.sync_copy(srcg_h.at[half, s], srcv)
        pltpu.sync_copy(dstl_h.at[half, s], dstv)

        @pl.when(half == 0)
        def _zero_items():
            pltpu.sync_copy(z375_h, acc_s.at[pl.ds(s * 375, 375)])

        @pl.when(half == 1)
        def _zero_users():
            pltpu.sync_copy(z375_h.at[pl.ds(0, 250)],
                            acc_s.at[pl.ds(s * 250, 250)])

        plsc.subcore_barrier()

        pltpu.async_copy(ysrc.at[srcv.at[0]], rowsA, semA)

        def _pair(k, carry):
            j0 = 2 * k
            j1 = j0 + 1
            pltpu.async_copy(ysrc.at[srcv.at[j1]], rowsB, semB)
            pltpu.make_async_copy(ysrc.at[srcv.at[j0]], rowsA, semA).wait()
            pltpu.sync_copy(rowsA, acc_s.at[dstv.at[j0]], add=True)

            @pl.when(j0 + 2 < NCHUNK)
            def _next():
                pltpu.async_copy(ysrc.at[srcv.at[j0 + 2]], rowsA, semA)

            pltpu.make_async_copy(ysrc.at[srcv.at[j1]], rowsB, semB).wait()
            pltpu.sync_copy(rowsB, acc_s.at[dstv.at[j1]], add=True)
            return carry
        lax.fori_loop(0, NCHUNK // 2, _pair, 0)
        pltpu.make_async_copy(ysrc.at[srcv.at[NCHUNK - 1]], rowsA, semA).wait()
        pltpu.sync_copy(rowsA, acc_s.at[dstv.at[NCHUNK - 1]], add=True)

        plsc.subcore_barrier()

        if t < 2:
            ydst = ys[t + 1]

            def _wb(base_l, base_g, nchunks, chrows):
                for kk in range(nchunks):
                    l0 = base_l + kk * chrows
                    g0 = base_g + kk * chrows
                    pltpu.sync_copy(acc_s.at[pl.ds(l0, chrows)],
                                    rowsA.at[pl.ds(0, chrows)])

                    def _r(rr, c2):
                        d16 = _splat(g0 + rr)
                        d2 = d16 * d16

                        def _k(kcol, c3):
                            sl = pl.ds(kcol * 16, 16)
                            rowsB[rr, sl] = rowsA[rr, sl] * d2
                            return c3
                        lax.fori_loop(0, 8, _k, 0)
                        return c2
                    lax.fori_loop(0, chrows, _r, 0)
                    pltpu.sync_copy(rowsB.at[pl.ds(0, chrows)],
                                    ydst.at[pl.ds(g0, chrows)])

            @pl.when(half == 0)
            def _wb_items():
                _wb(s * 375, NU + s * 375, 5, 75)

            @pl.when(half == 1)
            def _wb_users():
                _wb(s * 250, s * 250, 5, 50)

        def _ggather(g):
            pltpu.sync_copy(idxl_h.at[g, pl.ds(s * 4, 4)], gidx)
            for r in range(4):
                pltpu.async_copy(acc_s.at[gidx.at[r]],
                                 rowsA.at[pl.ds(0, 64)], semA).wait()
                pltpu.sync_copy(
                    rowsA.at[pl.ds(0, 64)],
                    G_h.at[(t + 1) * 3 + g, pl.ds(s * 256 + r * 64, 64)])

        @pl.when(half == 0)
        def _g_items():
            _ggather(1)
            _ggather(2)

        @pl.when(half == 1)
        def _g_users():
            _ggather(0)

        plsc.subcore_barrier()


_mega = functools.partial(
    pl.kernel,
    out_type=(jax.ShapeDtypeStruct((12, BATCH, K), jnp.float32),
              jax.ShapeDtypeStruct((3, 256, 16), jnp.float32),
              jax.ShapeDtypeStruct((NP, K), jnp.float32),
              jax.ShapeDtypeStruct((NP, K), jnp.float32),
              jax.ShapeDtypeStruct((NP, K), jnp.float32)),
    mesh=_mesh,
    compiler_params=_params,
    scratch_types=[
        pltpu.VMEM((NCHUNK, CH), jnp.int32),
        pltpu.VMEM((NCHUNK, CH), jnp.int32),
        pltpu.VMEM((CH, K), jnp.float32),
        pltpu.VMEM((CH, K), jnp.float32),
        pltpu.VMEM((NP // 16, 16), jnp.float32),
        pltpu.VMEM((5, 128), jnp.int32),
        pltpu.VMEM((NP // 16, 16), jnp.float32),
        pltpu.VMEM((40, 16), jnp.float32),
        pltpu.VMEM((4, 64), jnp.int32),
        pltpu.VMEM((16, 16), jnp.float32),
        pltpu.VMEM_SHARED((NI, K), jnp.float32),
        pltpu.VMEM_SHARED((NP // 16, 16), jnp.float32),
        pltpu.VMEM_SHARED((NP // 16, 16), jnp.float32),
        pltpu.SemaphoreType.DMA,
        pltpu.SemaphoreType.DMA,
    ],
)(_mega_body)


def _loss_body(g_ref, d_ref, o_ref):
    g = g_ref[...]
    dva = d_ref[...]
    du = dva[0][:, None]
    dp = dva[1][:, None]
    dn = dva[2][:, None]
    u = (g[0] + (g[3] + g[6] + g[9]) * du) * 0.25
    p = (g[1] + (g[4] + g[7] + g[10]) * dp) * 0.25
    nn = (g[2] + (g[5] + g[8] + g[11]) * dn) * 0.25
    xpos = jnp.sum(u * p, axis=1)
    xneg = jnp.sum(u * nn, axis=1)
    z = xneg - xpos
    sp = jnp.maximum(z, 0.0) + jnp.log1p(jnp.exp(-jnp.abs(z)))
    loss = jnp.mean(sp)
    reg = LW * 0.5 * (jnp.sum(g[0] ** 2) + jnp.sum(g[1] ** 2)
                      + jnp.sum(g[2] ** 2)) / BATCH
    o_ref[...] = jnp.reshape(loss + reg, (1, 1))


def _loss(G, dva):
    return pl.pallas_call(
        _loss_body,
        out_shape=jax.ShapeDtypeStruct((1, 1), jnp.float32),
    )(G, dva)


def kernel(Gu, Gi, edge_index, user, pos, neg):
    src = edge_index[0].astype(jnp.int32)
    dst = edge_index[1].astype(jnp.int32)
    srcg4 = src.reshape(2, NSUB, NCHUNK, CH)
    dstl4 = (dst.reshape(2, NSUB, NCHUNK, CH)
             - jnp.array([NU, 0], jnp.int32).reshape(2, 1, 1, 1))
    x0p = jnp.pad(jnp.concatenate([Gu, Gi], axis=0), ((0, NP - N), (0, 0)))
    zdeg = jnp.zeros((NP // 16, 16), jnp.float32)
    z375 = jnp.zeros((375, K), jnp.float32)
    iden = jnp.arange(NP // 16, dtype=jnp.int32).reshape(5, 128)
    u32 = user.astype(jnp.int32)
    p32 = pos.astype(jnp.int32)
    n32 = neg.astype(jnp.int32)
    idxg3 = jnp.stack([u32, NU + p32, NU + n32]).reshape(3, 64, 64)
    idxl3 = jnp.stack([u32, p32, n32]).reshape(3, 64, 64)
    G, dva, _y0, _y1, _y2 = _mega(srcg4, dstl4, x0p, zdeg, z375, iden,
                                  idxg3, idxl3)
    return _loss(G, dva.reshape(3, BATCH))[0, 0]


# speedup vs baseline: 1.0881x; 1.0881x over previous
import functools

import jax
import jax.numpy as jnp
from jax import lax
from jax.experimental import pallas as pl
from jax.experimental.pallas import tpu as pltpu
from jax.experimental.pallas import tpu_sc as plsc

NU, NI = 4000, 6000
N = NU + NI
NP = 10240
K = 128
NE = 320000
NSUB = 16
CH = 125
NCHUNK = (NE // (2 * NSUB)) // CH
BATCH = 4096
LW = 1e-4

_mesh = plsc.VectorSubcoreMesh(core_axis_name="c", subcore_axis_name="s")
_params = pltpu.CompilerParams(needs_layout_passes=False,
                               use_tc_tiling_on_sc=False)


def _vec(v, dtype=jnp.float32):
    return jnp.full((16,), v, dtype)


def _dinv16(d):
    x = jnp.maximum(d, _vec(1.0))
    i = lax.bitcast_convert_type(x, jnp.int32)
    i = _vec(0x5F3759DF, jnp.int32) - lax.shift_right_logical(i, _vec(1, jnp.int32))
    y = lax.bitcast_convert_type(i, jnp.float32)
    half = _vec(0.5) * x
    c15 = _vec(1.5)
    for _ in range(3):
        y = y * (c15 - half * y * y)
    return jnp.where(d > _vec(0.0), y, _vec(0.0))


def _mega_body(srcg_h, dstl_h, x0_h, zdeg_h, z375_h, iden_h, idxg_h, idxl_h,
               G_h, dva_h, y0_h, y1_h, y2_h,
               srcv, dstv, rowsA, rowsB, hist_v, iden_v, dinvv, deg40,
               gidx, dvbuf, acc_s, deg_s, dinv_s, semA, semB):
    c = lax.axis_index("c")
    s = lax.axis_index("s")
    pltpu.sync_copy(iden_h, iden_v)

    def _splat(w):
        return plsc.load_gather(
            dinvv, [jnp.full((16,), w // 16, jnp.int32),
                    jnp.full((16,), w % 16, jnp.int32)])

    def _z(i, carry):
        hist_v[i] = jnp.zeros((16,), jnp.float32)
        return carry
    lax.fori_loop(0, NP // 16, _z, 0)

    ones = _vec(1.0)
    for half in range(2):
        pltpu.sync_copy(dstl_h.at[half, s], dstv)
        off = _vec(NU if half == 0 else 0, jnp.int32)

        def _h(i, carry):
            p16 = jnp.full((16,), i * 16, jnp.int32) + lax.iota(jnp.int32, 16)
            r = p16 // _vec(CH, jnp.int32)
            col = p16 - r * _vec(CH, jnp.int32)
            idx = plsc.load_gather(dstv, [r, col]) + off
            hi = lax.shift_right_logical(idx, _vec(4, jnp.int32))
            lo = jnp.bitwise_and(idx, _vec(15, jnp.int32))
            plsc.addupdate_scatter(hist_v, [hi, lo], ones)
            return carry
        lax.fori_loop(0, NCHUNK * CH // 16, _h, 0)

    pltpu.sync_copy(zdeg_h.at[pl.ds(s * 40, 40)], deg_s.at[pl.ds(s * 40, 40)])
    plsc.subcore_barrier()
    for r in range(5):
        pltpu.sync_copy(hist_v.at[pl.ds(r * 128, 128)],
                        deg_s.at[iden_v.at[r]], add=True)
    plsc.subcore_barrier()

    pltpu.sync_copy(deg_s.at[pl.ds(s * 40, 40)], deg40)

    def _dv(i, carry):
        deg40[i] = _dinv16(deg40[i])
        return carry
    lax.fori_loop(0, 40, _dv, 0)
    pltpu.sync_copy(deg40, dinv_s.at[pl.ds(s * 40, 40)])
    plsc.subcore_barrier()
    pltpu.sync_copy(dinv_s, dinvv)

    def _dvals(g):
        pltpu.sync_copy(idxg_h.at[g, pl.ds(s * 4, 4)], gidx)

        def _dvb(rk, carry):
            i16 = gidx[rk // 4, pl.ds((rk % 4) * 16, 16)]
            hi = lax.shift_right_logical(i16, _vec(4, jnp.int32))
            lo = jnp.bitwise_and(i16, _vec(15, jnp.int32))
            dvbuf[rk] = plsc.load_gather(dinvv, [hi, lo])
            return carry
        lax.fori_loop(0, 16, _dvb, 0)
        pltpu.sync_copy(dvbuf, dva_h.at[g, pl.ds(s * 16, 16)])

    @pl.when(c == 0)
    def _dvals_items():
        _dvals(1)
        _dvals(2)

    @pl.when(c == 1)
    def _dvals_users():
        _dvals(0)

    for g in range(3):
        pltpu.sync_copy(idxg_h.at[g, pl.ds(c * 32 + s * 2, 2)],
                        gidx.at[pl.ds(0, 2)])
        for kk in range(2):
            pltpu.async_copy(x0_h.at[gidx.at[kk]],
                             rowsA.at[pl.ds(0, 64)], semA).wait()
            pltpu.sync_copy(rowsA.at[pl.ds(0, 64)],
                            G_h.at[g, pl.ds(c * 2048 + s * 128 + kk * 64, 64)])

    def _scale_rows(src_h, dst_h, base, nchunks, chrows):
        for kk in range(nchunks):
            row0 = base + kk * chrows
            pltpu.sync_copy(src_h.at[pl.ds(row0, chrows)],
                            rowsA.at[pl.ds(0, chrows)])

            def _r(rr, c2):
                d16 = _splat(row0 + rr)
                for kcol in range(8):
                    sl = pl.ds(kcol * 16, 16)
                    rowsB[rr, sl] = rowsA[rr, sl] * d16
                return c2
            lax.fori_loop(0, chrows, _r, 0)
            pltpu.sync_copy(rowsB.at[pl.ds(0, chrows)],
                            dst_h.at[pl.ds(row0, chrows)])

    @pl.when(c == 0)
    def _y0_users():
        _scale_rows(x0_h, y0_h, s * 250, 2, 125)

    @pl.when(c == 1)
    def _y0_items():
        _scale_rows(x0_h, y0_h, NU + s * 375, 3, 125)

    plsc.subcore_barrier()

    ys = (y0_h, y1_h, y2_h)
    for t in range(3):
        ysrc = ys[t]

        half = (t + c) % 2
        pltpu.sync_copy(srcg_h.at[half, s], srcv)
        pltpu.sync_copy(dstl_h.at[half, s], dstv)

        @pl.when(half == 0)
        def _zero_items():
            pltpu.sync_copy(z375_h, acc_s.at[pl.ds(s * 375, 375)])

        @pl.when(half == 1)
        def _zero_users():
            pltpu.sync_copy(z375_h.at[pl.ds(0, 250)],
                            acc_s.at[pl.ds(s * 250, 250)])

        plsc.subcore_barrier()

        pltpu.async_copy(ysrc.at[srcv.at[0]], rowsA, semA)

        def _pair(k, carry):
            j0 = 2 * k
            j1 = j0 + 1
            pltpu.async_copy(ysrc.at[srcv.at[j1]], rowsB, semB)
            pltpu.make_async_copy(ysrc.at[srcv.at[j0]], rowsA, semA).wait()
            pltpu.sync_copy(rowsA, acc_s.at[dstv.at[j0]], add=True)

            @pl.when(j0 + 2 < NCHUNK)
            def _next():
                pltpu.async_copy(ysrc.at[srcv.at[j0 + 2]], rowsA, semA)

            pltpu.make_async_copy(ysrc.at[srcv.at[j1]], rowsB, semB).wait()
            pltpu.sync_copy(rowsB, acc_s.at[dstv.at[j1]], add=True)
            return carry
        lax.fori_loop(0, NCHUNK // 2, _pair, 0)

        plsc.subcore_barrier()

        if t < 2:
            ydst = ys[t + 1]

            def _wb(base_l, base_g, nchunks, chrows):
                for kk in range(nchunks):
                    l0 = base_l + kk * chrows
                    g0 = base_g + kk * chrows
                    pltpu.sync_copy(acc_s.at[pl.ds(l0, chrows)],
                                    rowsA.at[pl.ds(0, chrows)])

                    def _r(rr, c2):
                        d16 = _splat(g0 + rr)
                        d2 = d16 * d16
                        for kcol in range(8):
                            sl = pl.ds(kcol * 16, 16)
                            rowsB[rr, sl] = rowsA[rr, sl] * d2
                        return c2
                    lax.fori_loop(0, chrows, _r, 0)
                    pltpu.sync_copy(rowsB.at[pl.ds(0, chrows)],
                                    ydst.at[pl.ds(g0, chrows)])

            @pl.when(half == 0)
            def _wb_items():
                _wb(s * 375, NU + s * 375, 3, 125)

            @pl.when(half == 1)
            def _wb_users():
                _wb(s * 250, s * 250, 2, 125)

        def _ggather(g):
            pltpu.sync_copy(idxl_h.at[g, pl.ds(s * 4, 4)], gidx)
            for r in range(4):
                pltpu.async_copy(acc_s.at[gidx.at[r]],
                                 rowsA.at[pl.ds(0, 64)], semA).wait()
                pltpu.sync_copy(
                    rowsA.at[pl.ds(0, 64)],
                    G_h.at[(t + 1) * 3 + g, pl.ds(s * 256 + r * 64, 64)])

        @pl.when(half == 0)
        def _g_items():
            _ggather(1)
            _ggather(2)

        @pl.when(half == 1)
        def _g_users():
            _ggather(0)

        plsc.subcore_barrier()


_mega = functools.partial(
    pl.kernel,
    out_type=(jax.ShapeDtypeStruct((12, BATCH, K), jnp.float32),
              jax.ShapeDtypeStruct((3, 256, 16), jnp.float32),
              jax.ShapeDtypeStruct((NP, K), jnp.float32),
              jax.ShapeDtypeStruct((NP, K), jnp.float32),
              jax.ShapeDtypeStruct((NP, K), jnp.float32)),
    mesh=_mesh,
    compiler_params=_params,
    scratch_types=[
        pltpu.VMEM((NCHUNK, CH), jnp.int32),
        pltpu.VMEM((NCHUNK, CH), jnp.int32),
        pltpu.VMEM((CH, K), jnp.float32),
        pltpu.VMEM((CH, K), jnp.float32),
        pltpu.VMEM((NP // 16, 16), jnp.float32),
        pltpu.VMEM((5, 128), jnp.int32),
        pltpu.VMEM((NP // 16, 16), jnp.float32),
        pltpu.VMEM((40, 16), jnp.float32),
        pltpu.VMEM((4, 64), jnp.int32),
        pltpu.VMEM((16, 16), jnp.float32),
        pltpu.VMEM_SHARED((NI, K), jnp.float32),
        pltpu.VMEM_SHARED((NP // 16, 16), jnp.float32),
        pltpu.VMEM_SHARED((NP // 16, 16), jnp.float32),
        pltpu.SemaphoreType.DMA,
        pltpu.SemaphoreType.DMA,
    ],
)(_mega_body)


def _loss_body(g_ref, d_ref, o_ref):
    g = g_ref[...]
    dva = d_ref[...]
    du = dva[0][:, None]
    dp = dva[1][:, None]
    dn = dva[2][:, None]
    u = (g[0] + (g[3] + g[6] + g[9]) * du) * 0.25
    p = (g[1] + (g[4] + g[7] + g[10]) * dp) * 0.25
    nn = (g[2] + (g[5] + g[8] + g[11]) * dn) * 0.25
    xpos = jnp.sum(u * p, axis=1)
    xneg = jnp.sum(u * nn, axis=1)
    z = xneg - xpos
    sp = jnp.maximum(z, 0.0) + jnp.log1p(jnp.exp(-jnp.abs(z)))
    loss = jnp.mean(sp)
    reg = LW * 0.5 * (jnp.sum(g[0] ** 2) + jnp.sum(g[1] ** 2)
                      + jnp.sum(g[2] ** 2)) / BATCH
    o_ref[...] = jnp.reshape(loss + reg, (1, 1))


def _loss(G, dva):
    return pl.pallas_call(
        _loss_body,
        out_shape=jax.ShapeDtypeStruct((1, 1), jnp.float32),
    )(G, dva)


def kernel(Gu, Gi, edge_index, user, pos, neg):
    src = edge_index[0].astype(jnp.int32)
    dst = edge_index[1].astype(jnp.int32)
    srcg4 = src.reshape(2, NSUB, NCHUNK, CH)
    dstl4 = (dst.reshape(2, NSUB, NCHUNK, CH)
             - jnp.array([NU, 0], jnp.int32).reshape(2, 1, 1, 1))
    x0p = jnp.pad(jnp.concatenate([Gu, Gi], axis=0), ((0, NP - N), (0, 0)))
    zdeg = jnp.zeros((NP // 16, 16), jnp.float32)
    z375 = jnp.zeros((375, K), jnp.float32)
    iden = jnp.arange(NP // 16, dtype=jnp.int32).reshape(5, 128)
    u32 = user.astype(jnp.int32)
    p32 = pos.astype(jnp.int32)
    n32 = neg.astype(jnp.int32)
    idxg3 = jnp.stack([u32, NU + p32, NU + n32]).reshape(3, 64, 64)
    idxl3 = jnp.stack([u32, p32, n32]).reshape(3, 64, 64)
    G, dva, _y0, _y1, _y2 = _mega(srcg4, dstl4, x0p, zdeg, z375, iden,
                                  idxg3, idxl3)
    return _loss(G, dva.reshape(3, BATCH))[0, 0]
